# baseline (speedup 1.0000x reference)
import functools

import jax
import jax.numpy as jnp
from jax import lax
from jax.experimental import pallas as pl
from jax.experimental.pallas import tpu as pltpu
from jax.experimental.pallas import tpu_sc as plsc

N = 10000
E = 320000
NC = 2
NS = 16
NW = NC * NS
CH = 128
EW = E // NW
NCB = 78
TAIL = EW - NCB * CH
EMAIN = NW * NCB * CH
NBUF = 6
AHEAD = 3
NP = 10240
ROWS_PER_SUB = NP // NS
ZCOPIES = ROWS_PER_SUB // CH


def _sc_segment_sum(y, srcm, srct, dstm, dstt):
    D = y.shape[1]
    mesh = plsc.VectorSubcoreMesh(core_axis_name="c", subcore_axis_name="s")

    @functools.partial(
        pl.kernel,
        out_type=jax.ShapeDtypeStruct((NC, NP, D), jnp.float32),
        mesh=mesh,
        scratch_types=[
            pltpu.VMEM((NCB, CH), jnp.int32),
            pltpu.VMEM((NCB, CH), jnp.int32),
            pltpu.VMEM((1, TAIL), jnp.int32),
            pltpu.VMEM((1, TAIL), jnp.int32),
        ] + [pltpu.VMEM((CH, D), jnp.float32) for _ in range(NBUF)] + [
            pltpu.VMEM((TAIL, D), jnp.float32),
            pltpu.VMEM_SHARED((NP, D), jnp.float32),
            pltpu.SemaphoreType.DMA((NBUF,)),
            pltpu.SemaphoreType.DMA((NBUF,)),
        ],
        compiler_params=pltpu.CompilerParams(use_tc_tiling_on_sc=False),
    )
    def seg_kernel(y_hbm, srcm_hbm, srct_hbm, dstm_hbm, dstt_hbm, out_hbm,
                   src_v, dst_v, srct_v, dstt_v,
                   b0, b1, b2, b3, b4, b5, bt,
                   acc_sh, gsem, ssem):
        c = lax.axis_index("c")
        s = lax.axis_index("s")
        w = c * NS + s
        bufs = (b0, b1, b2, b3, b4, b5)

        pltpu.sync_copy(srcm_hbm.at[w], src_v)
        pltpu.sync_copy(dstm_hbm.at[w], dst_v)
        pltpu.sync_copy(srct_hbm.at[w], srct_v)
        pltpu.sync_copy(dstt_hbm.at[w], dstt_v)

        @pl.loop(0, CH)
        def _(i):
            for col in range(D // 16):
                b0[i, pl.ds(col * 16, 16)] = jnp.zeros((16,), jnp.float32)

        for r in range(ZCOPIES):
            pltpu.sync_copy(
                b0, acc_sh.at[pl.ds(s * ROWS_PER_SUB + r * CH, CH)])
        plsc.subcore_barrier()

        def g_start(ci, k):
            pltpu.async_copy(y_hbm.at[src_v.at[ci]], bufs[k], gsem.at[k])

        def g_wait(ci, k):
            pltpu.make_async_copy(y_hbm.at[src_v.at[ci]], bufs[k],
                                  gsem.at[k]).wait()

        def s_start(ci, k):
            pltpu.async_copy(bufs[k], acc_sh.at[dst_v.at[ci]], ssem.at[k],
                             add=True)

        def s_wait(ci, k):
            pltpu.make_async_copy(bufs[k], acc_sh.at[dst_v.at[ci]],
                                  ssem.at[k]).wait()

        for k in range(AHEAD):
            g_start(k, k)
        for k in range(NBUF):
            g_wait(k, k)
            s_start(k, k)
            if k >= AHEAD:
                s_wait(k - AHEAD, (k + AHEAD) % NBUF)
            g_start(k + AHEAD, (k + AHEAD) % NBUF)

        @pl.loop(NBUF, NCB - NBUF, step=NBUF)
        def _(j):
            for k in range(NBUF):
                ci = j + k
                g_wait(ci, k)
                s_start(ci, k)
                s_wait(ci - AHEAD, (k + AHEAD) % NBUF)
                g_start(ci + AHEAD, (k + AHEAD) % NBUF)

        for k in range(NBUF):
            ci = NCB - NBUF + k
            g_wait(ci, k)
            s_start(ci, k)
            s_wait(ci - AHEAD, (k + AHEAD) % NBUF)
            if k < AHEAD:
                g_start(ci + AHEAD, (k + AHEAD) % NBUF)
        for k in range(AHEAD, NBUF):
            s_wait(NCB - NBUF + k, k)

        pltpu.async_copy(y_hbm.at[srct_v.at[0]], bt, gsem.at[0]).wait()
        pltpu.sync_copy(bt, acc_sh.at[dstt_v.at[0]], add=True)

        plsc.subcore_barrier()
        pltpu.sync_copy(acc_sh.at[pl.ds(s * ROWS_PER_SUB, ROWS_PER_SUB)],
                        out_hbm.at[c, pl.ds(s * ROWS_PER_SUB, ROWS_PER_SUB)])

    return seg_kernel(y, srcm, srct, dstm, dstt)


def _dot_t(a, w):
    return lax.dot_general(a, w, (((1,), (1,)), ((), ())),
                           preferred_element_type=jnp.float32)


def _tc_pre(x, W_rel1, W_root1):
    def body(x_ref, wr_ref, wo_ref, y_ref, xr_ref):
        xv = x_ref[...]
        y_ref[...] = _dot_t(xv, wr_ref[...])
        xr_ref[...] = _dot_t(xv, wo_ref[...])

    return pl.pallas_call(
        body,
        out_shape=[jax.ShapeDtypeStruct((N, W_rel1.shape[0]), jnp.float32),
                   jax.ShapeDtypeStruct((N, W_root1.shape[0]), jnp.float32)],
    )(x, W_rel1, W_root1)


def _tc_mid(part1, xr1, b_rel1, W_l1, b_l1, W_rel2, W_root2, b_rel2):
    def body(p_ref, xr_ref, br1_ref, wl1_ref, bl1_ref, wr2_ref, wo2_ref,
             br2_ref, y2_ref, hr2_ref):
        h1 = p_ref[0, :N] + p_ref[1, :N] + xr_ref[...] + br1_ref[...]
        h2 = jnp.maximum(_dot_t(h1, wl1_ref[...]) + bl1_ref[...], 0.0)
        y2_ref[...] = _dot_t(h2, wr2_ref[...])
        hr2_ref[...] = _dot_t(h2, wo2_ref[...]) + br2_ref[...]

    return pl.pallas_call(
        body,
        out_shape=[jax.ShapeDtypeStruct((N, W_rel2.shape[0]), jnp.float32),
                   jax.ShapeDtypeStruct((N, W_root2.shape[0]), jnp.float32)],
    )(part1, xr1, b_rel1.reshape(1, -1), W_l1, b_l1.reshape(1, -1),
      W_rel2, W_root2, b_rel2.reshape(1, -1))


def _tc_post(part2, hr2, W_l2, b_l2):
    def body(p_ref, hr_ref, wl2_ref, bl2_ref, o_ref):
        h3 = p_ref[0, :N] + p_ref[1, :N] + hr_ref[...]
        logits = _dot_t(h3, wl2_ref[...]) + bl2_ref[...]
        m = jnp.max(logits, axis=1, keepdims=True)
        shifted = logits - m
        lse = jnp.log(jnp.sum(jnp.exp(shifted), axis=1, keepdims=True))
        o_ref[...] = shifted - lse

    return pl.pallas_call(
        body,
        out_shape=jax.ShapeDtypeStruct((N, W_l2.shape[0]), jnp.float32),
    )(part2, hr2, W_l2, b_l2.reshape(1, -1))


def kernel(x, edge_index, W_rel1, b_rel1, W_root1, W_l1, b_l1,
           W_rel2, b_rel2, W_root2, W_l2, b_l2):
    src = edge_index[0]
    dst = edge_index[1]
    srcm = src[:EMAIN].reshape(NW, NCB, CH)
    dstm = dst[:EMAIN].reshape(NW, NCB, CH)
    srct = src[EMAIN:].reshape(NW, 1, TAIL)
    dstt = dst[EMAIN:].reshape(NW, 1, TAIL)

    y1, xr1 = _tc_pre(x, W_rel1, W_root1)
    part1 = _sc_segment_sum(y1, srcm, srct, dstm, dstt)
    y2, hr2 = _tc_mid(part1, xr1, b_rel1, W_l1, b_l1, W_rel2, W_root2, b_rel2)
    part2 = _sc_segment_sum(y2, srcm, srct, dstm, dstt)
    return _tc_post(part2, hr2, W_l2, b_l2)


# speedup vs baseline: 1.0344x; 1.0344x over previous
import functools

import jax
import jax.numpy as jnp
from jax import lax
from jax.experimental import pallas as pl
from jax.experimental.pallas import tpu as pltpu
from jax.experimental.pallas import tpu_sc as plsc

N = 10000
E = 320000
NC = 2
NS = 16
NW = NC * NS
CH = 128
EW = E // NW
NCB = 78
TAIL = EW - NCB * CH
EMAIN = NW * NCB * CH
NBUF = 8
AHEAD = 4
NP = 10240
ROWS_PER_SUB = NP // NS
ZCOPIES = ROWS_PER_SUB // CH


def _sc_segment_sum(y, srcm, srct, dstm, dstt):
    D = y.shape[1]
    mesh = plsc.VectorSubcoreMesh(core_axis_name="c", subcore_axis_name="s")

    @functools.partial(
        pl.kernel,
        out_type=jax.ShapeDtypeStruct((NC, NP, D), jnp.float32),
        mesh=mesh,
        scratch_types=[
            pltpu.VMEM((NCB, CH), jnp.int32),
            pltpu.VMEM((NCB, CH), jnp.int32),
            pltpu.VMEM((1, TAIL), jnp.int32),
            pltpu.VMEM((1, TAIL), jnp.int32),
        ] + [pltpu.VMEM((CH, D), jnp.float32) for _ in range(NBUF)] + [
            pltpu.VMEM((TAIL, D), jnp.float32),
            pltpu.VMEM_SHARED((NP, D), jnp.float32),
            pltpu.SemaphoreType.DMA((NBUF,)),
            pltpu.SemaphoreType.DMA((NBUF,)),
        ],
        compiler_params=pltpu.CompilerParams(use_tc_tiling_on_sc=False),
    )
    def seg_kernel(y_hbm, srcm_hbm, srct_hbm, dstm_hbm, dstt_hbm, out_hbm,
                   src_v, dst_v, srct_v, dstt_v,
                   b0, b1, b2, b3, b4, b5, b6, b7, bt,
                   acc_sh, gsem, ssem):
        c = lax.axis_index("c")
        s = lax.axis_index("s")
        w = c * NS + s
        bufs = (b0, b1, b2, b3, b4, b5, b6, b7)

        pltpu.sync_copy(srcm_hbm.at[pl.ds(w * NCB, NCB)], src_v)
        pltpu.sync_copy(dstm_hbm.at[pl.ds(w * NCB, NCB)], dst_v)
        pltpu.sync_copy(srct_hbm.at[w], srct_v)
        pltpu.sync_copy(dstt_hbm.at[w], dstt_v)

        @pl.loop(0, CH)
        def _(i):
            for col in range(D // 16):
                b0[i, pl.ds(col * 16, 16)] = jnp.zeros((16,), jnp.float32)

        for r in range(ZCOPIES):
            pltpu.sync_copy(
                b0, acc_sh.at[pl.ds(s * ROWS_PER_SUB + r * CH, CH)])
        plsc.subcore_barrier()

        def g_start(ci, k):
            pltpu.async_copy(y_hbm.at[src_v.at[ci]], bufs[k], gsem.at[k])

        def g_wait(ci, k):
            pltpu.make_async_copy(y_hbm.at[src_v.at[ci]], bufs[k],
                                  gsem.at[k]).wait()

        def s_start(ci, k):
            pltpu.async_copy(bufs[k], acc_sh.at[dst_v.at[ci]], ssem.at[k],
                             add=True)

        def s_wait(ci, k):
            pltpu.make_async_copy(bufs[k], acc_sh.at[dst_v.at[ci]],
                                  ssem.at[k]).wait()

        for k in range(AHEAD):
            g_start(k, k)
        for k in range(NBUF):
            g_wait(k, k)
            s_start(k, k)
            if k >= AHEAD:
                s_wait(k - AHEAD, (k + AHEAD) % NBUF)
            g_start(k + AHEAD, (k + AHEAD) % NBUF)

        @pl.loop(NBUF, 72, step=NBUF)
        def _(j):
            for k in range(NBUF):
                ci = j + k
                g_wait(ci, k)
                s_start(ci, k)
                s_wait(ci - AHEAD, (k + AHEAD) % NBUF)
                g_start(ci + AHEAD, (k + AHEAD) % NBUF)

        for k in range(NCB - 72):
            ci = 72 + k
            g_wait(ci, k)
            s_start(ci, k)
            s_wait(ci - AHEAD, (k + AHEAD) % NBUF)
            if ci + AHEAD < NCB:
                g_start(ci + AHEAD, (k + AHEAD) % NBUF)
        for k in range(2, NCB - 72):
            s_wait(72 + k, k)

        pltpu.async_copy(y_hbm.at[srct_v.at[0]], bt, gsem.at[0]).wait()
        pltpu.sync_copy(bt, acc_sh.at[dstt_v.at[0]], add=True)

        plsc.subcore_barrier()
        pltpu.sync_copy(acc_sh.at[pl.ds(s * ROWS_PER_SUB, ROWS_PER_SUB)],
                        out_hbm.at[c, pl.ds(s * ROWS_PER_SUB, ROWS_PER_SUB)])

    return seg_kernel(y, srcm, srct, dstm, dstt)


def _dot_t(a, w):
    return lax.dot_general(a, w, (((1,), (1,)), ((), ())),
                           preferred_element_type=jnp.float32)


def _tc_pre(x, W_rel1, W_root1):
    def body(x_ref, wr_ref, wo_ref, y_ref, xr_ref):
        xv = x_ref[...]
        y_ref[...] = _dot_t(xv, wr_ref[...])
        xr_ref[...] = _dot_t(xv, wo_ref[...])

    return pl.pallas_call(
        body,
        out_shape=[jax.ShapeDtypeStruct((N, W_rel1.shape[0]), jnp.float32),
                   jax.ShapeDtypeStruct((N, W_root1.shape[0]), jnp.float32)],
    )(x, W_rel1, W_root1)


def _tc_mid(part1, xr1, b_rel1, W_l1, b_l1, W_rel2, W_root2, b_rel2):
    def body(p_ref, xr_ref, br1_ref, wl1_ref, bl1_ref, wr2_ref, wo2_ref,
             br2_ref, y2_ref, hr2_ref):
        h1 = p_ref[0, :N] + p_ref[1, :N] + xr_ref[...] + br1_ref[...]
        h2 = jnp.maximum(_dot_t(h1, wl1_ref[...]) + bl1_ref[...], 0.0)
        y2_ref[...] = _dot_t(h2, wr2_ref[...])
        hr2_ref[...] = _dot_t(h2, wo2_ref[...]) + br2_ref[...]

    return pl.pallas_call(
        body,
        out_shape=[jax.ShapeDtypeStruct((N, W_rel2.shape[0]), jnp.float32),
                   jax.ShapeDtypeStruct((N, W_root2.shape[0]), jnp.float32)],
    )(part1, xr1, b_rel1.reshape(1, -1), W_l1, b_l1.reshape(1, -1),
      W_rel2, W_root2, b_rel2.reshape(1, -1))


def _tc_post(part2, hr2, W_l2, b_l2):
    def body(p_ref, hr_ref, wl2_ref, bl2_ref, o_ref):
        h3 = p_ref[0, :N] + p_ref[1, :N] + hr_ref[...]
        logits = _dot_t(h3, wl2_ref[...]) + bl2_ref[...]
        m = jnp.max(logits, axis=1, keepdims=True)
        shifted = logits - m
        lse = jnp.log(jnp.sum(jnp.exp(shifted), axis=1, keepdims=True))
        o_ref[...] = shifted - lse

    return pl.pallas_call(
        body,
        out_shape=jax.ShapeDtypeStruct((N, W_l2.shape[0]), jnp.float32),
    )(part2, hr2, W_l2, b_l2.reshape(1, -1))


def kernel(x, edge_index, W_rel1, b_rel1, W_root1, W_l1, b_l1,
           W_rel2, b_rel2, W_root2, W_l2, b_l2):
    src = edge_index[0]
    dst = edge_index[1]
    srcm = src[:EMAIN].reshape(NW * NCB, CH)
    dstm = dst[:EMAIN].reshape(NW * NCB, CH)
    srct = src[EMAIN:].reshape(NW, 1, TAIL)
    dstt = dst[EMAIN:].reshape(NW, 1, TAIL)

    y1, xr1 = _tc_pre(x, W_rel1, W_root1)
    part1 = _sc_segment_sum(y1, srcm, srct, dstm, dstt)
    y2, hr2 = _tc_mid(part1, xr1, b_rel1, W_l1, b_l1, W_rel2, W_root2, b_rel2)
    part2 = _sc_segment_sum(y2, srcm, srct, dstm, dstt)
    return _tc_post(part2, hr2, W_l2, b_l2)


# speedup vs baseline: 1.2753x; 1.2329x over previous
import functools

import jax
import jax.numpy as jnp
from jax import lax
from jax.experimental import pallas as pl
from jax.experimental.pallas import tpu as pltpu
from jax.experimental.pallas import tpu_sc as plsc

N = 10000
E = 320000
NC = 2
NS = 16
NW = NC * NS
CH = 128
EW = E // NW
NCB = EW // CH
TAIL = EW - NCB * CH
NBUF = 6
AHEAD = 3
NP = 10240
ROWS_PER_SUB = NP // NS
ZCOPIES = ROWS_PER_SUB // CH


def _sc_segment_sum(y, edge):
    D = y.shape[1]
    mesh = plsc.VectorSubcoreMesh(core_axis_name="c", subcore_axis_name="s")

    @functools.partial(
        pl.kernel,
        out_type=jax.ShapeDtypeStruct((NC, NP, 128), jnp.float32),
        mesh=mesh,
        scratch_types=[
            pltpu.VMEM((NCB, CH), jnp.int32),
            pltpu.VMEM((NCB, CH), jnp.int32),
            pltpu.VMEM((1, TAIL), jnp.int32),
            pltpu.VMEM((1, TAIL), jnp.int32),
        ] + [pltpu.VMEM((CH, D), jnp.float32) for _ in range(NBUF)] + [
            pltpu.VMEM((TAIL, D), jnp.float32),
            pltpu.VMEM((CH, 128), jnp.float32),
            pltpu.VMEM_SHARED((NP, D), jnp.float32),
            pltpu.SemaphoreType.DMA((NBUF,)),
            pltpu.SemaphoreType.DMA((NBUF,)),
            pltpu.SemaphoreType.DMA,
        ],
        compiler_params=pltpu.CompilerParams(use_tc_tiling_on_sc=False),
    )
    def seg_kernel(y_hbm, edge_hbm, out_hbm,
                   src2_v, dst2_v, srct_v, dstt_v,
                   b0, b1, b2, b3, b4, b5, bt, stag_v,
                   acc_sh, gsem, ssem, isem):
        c = lax.axis_index("c")
        s = lax.axis_index("s")
        w = c * NS + s
        bufs = (b0, b1, b2, b3, b4, b5)

        @pl.loop(0, NCB)
        def _(i):
            off = w * EW + i * CH
            pltpu.async_copy(edge_hbm.at[0, pl.ds(off, CH)], src2_v.at[i],
                             isem)
            pltpu.async_copy(edge_hbm.at[1, pl.ds(off, CH)], dst2_v.at[i],
                             isem)

        toff = w * EW + NCB * CH
        pltpu.async_copy(edge_hbm.at[0, pl.ds(toff, TAIL)],
                         srct_v.at[0], isem)
        pltpu.async_copy(edge_hbm.at[1, pl.ds(toff, TAIL)],
                         dstt_v.at[0], isem)

        @pl.loop(0, CH)
        def _(i):
            for col in range(D // 16):
                b0[i, pl.ds(col * 16, 16)] = jnp.zeros((16,), jnp.float32)

        for r in range(ZCOPIES):
            pltpu.sync_copy(
                b0, acc_sh.at[pl.ds(s * ROWS_PER_SUB + r * CH, CH)])

        @pl.loop(0, NCB)
        def _(i):
            pltpu.make_async_copy(edge_hbm.at[0, pl.ds(w * EW, CH)],
                                  src2_v.at[0], isem).wait()
            pltpu.make_async_copy(edge_hbm.at[1, pl.ds(w * EW, CH)],
                                  dst2_v.at[0], isem).wait()

        pltpu.make_async_copy(edge_hbm.at[0, pl.ds(toff, TAIL)],
                              srct_v.at[0], isem).wait()
        pltpu.make_async_copy(edge_hbm.at[1, pl.ds(toff, TAIL)],
                              dstt_v.at[0], isem).wait()
        plsc.subcore_barrier()

        def g_start(ci, k):
            pltpu.async_copy(y_hbm.at[src2_v.at[ci]], bufs[k], gsem.at[k])

        def g_wait(ci, k):
            pltpu.make_async_copy(y_hbm.at[src2_v.at[ci]], bufs[k],
                                  gsem.at[k]).wait()

        def s_start(ci, k):
            pltpu.async_copy(bufs[k], acc_sh.at[dst2_v.at[ci]], ssem.at[k],
                             add=True)

        def s_wait(ci, k):
            pltpu.make_async_copy(bufs[k], acc_sh.at[dst2_v.at[ci]],
                                  ssem.at[k]).wait()

        for k in range(AHEAD):
            g_start(k, k)
        for k in range(NBUF):
            g_wait(k, k)
            s_start(k, k)
            if k >= AHEAD:
                s_wait(k - AHEAD, (k + AHEAD) % NBUF)
            g_start(k + AHEAD, (k + AHEAD) % NBUF)

        @pl.loop(NBUF, NCB - NBUF, step=NBUF)
        def _(j):
            for k in range(NBUF):
                ci = j + k
                g_wait(ci, k)
                s_start(ci, k)
                s_wait(ci - AHEAD, (k + AHEAD) % NBUF)
                g_start(ci + AHEAD, (k + AHEAD) % NBUF)

        for k in range(NBUF):
            ci = NCB - NBUF + k
            g_wait(ci, k)
            s_start(ci, k)
            s_wait(ci - AHEAD, (k + AHEAD) % NBUF)
            if ci + AHEAD < NCB:
                g_start(ci + AHEAD, (k + AHEAD) % NBUF)
        for k in range(NBUF - AHEAD, NBUF):
            s_wait(NCB - NBUF + k, k)

        pltpu.async_copy(y_hbm.at[srct_v.at[0]], bt, gsem.at[0]).wait()
        pltpu.sync_copy(bt, acc_sh.at[dstt_v.at[0]], add=True)

        plsc.subcore_barrier()
        @pl.loop(0, ZCOPIES)
        def _(r):
            off = s * ROWS_PER_SUB + r * CH
            pltpu.sync_copy(acc_sh.at[pl.ds(off, CH)],
                            stag_v.at[pl.ds(0, CH), pl.ds(0, D)])
            pltpu.sync_copy(stag_v, out_hbm.at[c, pl.ds(off, CH)])

    return seg_kernel(y, edge)


def _dot_t(a, w):
    return lax.dot_general(a, w, (((1,), (1,)), ((), ())),
                           preferred_element_type=jnp.float32)


def _tc_pre(x, W_rel1, W_root1):
    def body(x_ref, wr_ref, wo_ref, y_ref, xr_ref):
        xv = x_ref[...]
        y_ref[...] = _dot_t(xv, wr_ref[...])
        xr_ref[...] = _dot_t(xv, wo_ref[...])

    return pl.pallas_call(
        body,
        out_shape=[jax.ShapeDtypeStruct((N, W_rel1.shape[0]), jnp.float32),
                   jax.ShapeDtypeStruct((N, W_root1.shape[0]), jnp.float32)],
    )(x, W_rel1, W_root1)


def _tc_mid(part1, xr1, b_rel1, W_l1, b_l1, W_rel2, W_root2, b_rel2):
    H1 = W_l1.shape[1]

    def body(p_ref, xr_ref, br1_ref, wl1_ref, bl1_ref, wr2_ref, wo2_ref,
             br2_ref, y2_ref, hr2_ref):
        h1 = (p_ref[0, :N, :H1] + p_ref[1, :N, :H1] + xr_ref[...]
              + br1_ref[...])
        h2 = jnp.maximum(_dot_t(h1, wl1_ref[...]) + bl1_ref[...], 0.0)
        y2_ref[...] = _dot_t(h2, wr2_ref[...])
        hr2_ref[...] = _dot_t(h2, wo2_ref[...]) + br2_ref[...]

    return pl.pallas_call(
        body,
        out_shape=[jax.ShapeDtypeStruct((N, W_rel2.shape[0]), jnp.float32),
                   jax.ShapeDtypeStruct((N, W_root2.shape[0]), jnp.float32)],
    )(part1, xr1, b_rel1.reshape(1, -1), W_l1, b_l1.reshape(1, -1),
      W_rel2, W_root2, b_rel2.reshape(1, -1))


def _tc_post(part2, hr2, W_l2, b_l2):
    H3 = W_l2.shape[1]

    def body(p_ref, hr_ref, wl2_ref, bl2_ref, o_ref):
        h3 = p_ref[0, :N, :H3] + p_ref[1, :N, :H3] + hr_ref[...]
        logits = _dot_t(h3, wl2_ref[...]) + bl2_ref[...]
        m = jnp.max(logits, axis=1, keepdims=True)
        shifted = logits - m
        lse = jnp.log(jnp.sum(jnp.exp(shifted), axis=1, keepdims=True))
        o_ref[...] = shifted - lse

    return pl.pallas_call(
        body,
        out_shape=jax.ShapeDtypeStruct((N, W_l2.shape[0]), jnp.float32),
    )(part2, hr2, W_l2, b_l2.reshape(1, -1))


def kernel(x, edge_index, W_rel1, b_rel1, W_root1, W_l1, b_l1,
           W_rel2, b_rel2, W_root2, W_l2, b_l2):
    y1, xr1 = _tc_pre(x, W_rel1, W_root1)
    part1 = _sc_segment_sum(y1, edge_index)
    y2, hr2 = _tc_mid(part1, xr1, b_rel1, W_l1, b_l1, W_rel2, W_root2, b_rel2)
    part2 = _sc_segment_sum(y2, edge_index)
    return _tc_post(part2, hr2, W_l2, b_l2)


# speedup vs baseline: 1.3125x; 1.0292x over previous
import functools

import jax
import jax.numpy as jnp
from jax import lax
from jax.experimental import pallas as pl
from jax.experimental.pallas import tpu as pltpu
from jax.experimental.pallas import tpu_sc as plsc

N = 10000
E = 320000
NC = 2
NS = 16
NW = NC * NS
CH = 128
EW = E // NW
NCB = EW // CH
TAIL = EW - NCB * CH
RING = {64: (6, 3), 16: (8, 4)}
NP = 10240
ROWS_PER_SUB = NP // NS
ZCOPIES = ROWS_PER_SUB // CH


def _sc_segment_sum(y, edge):
    D = y.shape[1]
    NBUF, AHEAD = RING[D]
    mesh = plsc.VectorSubcoreMesh(core_axis_name="c", subcore_axis_name="s")

    @functools.partial(
        pl.kernel,
        out_type=jax.ShapeDtypeStruct((NC, NP, 128), jnp.float32),
        mesh=mesh,
        scratch_types=[
            pltpu.VMEM((NCB, CH), jnp.int32),
            pltpu.VMEM((NCB, CH), jnp.int32),
            pltpu.VMEM((1, TAIL), jnp.int32),
            pltpu.VMEM((1, TAIL), jnp.int32),
        ] + [pltpu.VMEM((CH, D), jnp.float32) for _ in range(NBUF)] + [
            pltpu.VMEM((TAIL, D), jnp.float32),
            pltpu.VMEM((CH, 128), jnp.float32),
            pltpu.VMEM_SHARED((NP, D), jnp.float32),
            pltpu.SemaphoreType.DMA((NBUF,)),
            pltpu.SemaphoreType.DMA((NBUF,)),
            pltpu.SemaphoreType.DMA,
        ],
        compiler_params=pltpu.CompilerParams(use_tc_tiling_on_sc=False),
    )
    def seg_kernel(y_hbm, edge_hbm, out_hbm, src2_v, dst2_v, srct_v, dstt_v,
                   *rest):
        bufs = rest[:NBUF]
        bt, stag_v, acc_sh, gsem, ssem, isem = rest[NBUF:]
        c = lax.axis_index("c")
        s = lax.axis_index("s")
        w = c * NS + s

        @pl.loop(0, NCB)
        def _(i):
            off = w * EW + i * CH
            pltpu.async_copy(edge_hbm.at[0, pl.ds(off, CH)], src2_v.at[i],
                             isem)
            pltpu.async_copy(edge_hbm.at[1, pl.ds(off, CH)], dst2_v.at[i],
                             isem)

        toff = w * EW + NCB * CH
        pltpu.async_copy(edge_hbm.at[0, pl.ds(toff, TAIL)],
                         srct_v.at[0], isem)
        pltpu.async_copy(edge_hbm.at[1, pl.ds(toff, TAIL)],
                         dstt_v.at[0], isem)

        @pl.loop(0, CH)
        def _(i):
            for col in range(D // 16):
                bufs[0][i, pl.ds(col * 16, 16)] = jnp.zeros((16,),
                                                            jnp.float32)

        for r in range(ZCOPIES):
            pltpu.sync_copy(
                bufs[0], acc_sh.at[pl.ds(s * ROWS_PER_SUB + r * CH, CH)])

        @pl.loop(0, NCB)
        def _(i):
            pltpu.make_async_copy(edge_hbm.at[0, pl.ds(w * EW, CH)],
                                  src2_v.at[0], isem).wait()
            pltpu.make_async_copy(edge_hbm.at[1, pl.ds(w * EW, CH)],
                                  dst2_v.at[0], isem).wait()

        pltpu.make_async_copy(edge_hbm.at[0, pl.ds(toff, TAIL)],
                              srct_v.at[0], isem).wait()
        pltpu.make_async_copy(edge_hbm.at[1, pl.ds(toff, TAIL)],
                              dstt_v.at[0], isem).wait()
        plsc.subcore_barrier()

        def g_start(ci, k):
            pltpu.async_copy(y_hbm.at[src2_v.at[ci]], bufs[k], gsem.at[k])

        def g_wait(ci, k):
            pltpu.make_async_copy(y_hbm.at[src2_v.at[ci]], bufs[k],
                                  gsem.at[k]).wait()

        def s_start(ci, k):
            pltpu.async_copy(bufs[k], acc_sh.at[dst2_v.at[ci]], ssem.at[k],
                             add=True)

        def s_wait(ci, k):
            pltpu.make_async_copy(bufs[k], acc_sh.at[dst2_v.at[ci]],
                                  ssem.at[k]).wait()

        for k in range(AHEAD):
            g_start(k, k)
        for k in range(NBUF):
            g_wait(k, k)
            s_start(k, k)
            if k >= AHEAD:
                s_wait(k - AHEAD, (k + AHEAD) % NBUF)
            g_start(k + AHEAD, (k + AHEAD) % NBUF)

        EP = NCB % NBUF if NCB % NBUF >= AHEAD else NCB % NBUF + NBUF

        @pl.loop(NBUF, NCB - EP, step=NBUF)
        def _(j):
            for k in range(NBUF):
                ci = j + k
                g_wait(ci, k)
                s_start(ci, k)
                s_wait(ci - AHEAD, (k + AHEAD) % NBUF)
                g_start(ci + AHEAD, (k + AHEAD) % NBUF)

        for k in range(EP):
            ci = NCB - EP + k
            g_wait(ci, k)
            s_start(ci, k)
            s_wait(ci - AHEAD, (k + AHEAD) % NBUF)
            if ci + AHEAD < NCB:
                g_start(ci + AHEAD, (k + AHEAD) % NBUF)
        for k in range(EP - AHEAD, EP):
            s_wait(NCB - EP + k, k)

        pltpu.async_copy(y_hbm.at[srct_v.at[0]], bt, gsem.at[0]).wait()
        pltpu.sync_copy(bt, acc_sh.at[dstt_v.at[0]], add=True)

        plsc.subcore_barrier()
        @pl.loop(0, ZCOPIES)
        def _(r):
            off = s * ROWS_PER_SUB + r * CH
            pltpu.sync_copy(acc_sh.at[pl.ds(off, CH)],
                            stag_v.at[pl.ds(0, CH), pl.ds(0, D)])
            pltpu.sync_copy(stag_v, out_hbm.at[c, pl.ds(off, CH)])

    return seg_kernel(y, edge)


def _dot_t(a, w):
    return lax.dot_general(a, w, (((1,), (1,)), ((), ())),
                           preferred_element_type=jnp.float32)


def _tc_pre(x, W_rel1, W_root1):
    def body(x_ref, wr_ref, wo_ref, y_ref, xr_ref):
        xv = x_ref[...]
        y_ref[...] = _dot_t(xv, wr_ref[...])
        xr_ref[...] = _dot_t(xv, wo_ref[...])

    return pl.pallas_call(
        body,
        out_shape=[jax.ShapeDtypeStruct((N, W_rel1.shape[0]), jnp.float32),
                   jax.ShapeDtypeStruct((N, W_root1.shape[0]), jnp.float32)],
    )(x, W_rel1, W_root1)


def _tc_mid(part1, xr1, b_rel1, W_l1, b_l1, W_rel2, W_root2, b_rel2):
    H1 = W_l1.shape[1]

    def body(p_ref, xr_ref, br1_ref, wl1_ref, bl1_ref, wr2_ref, wo2_ref,
             br2_ref, y2_ref, hr2_ref):
        h1 = (p_ref[0, :N, :H1] + p_ref[1, :N, :H1] + xr_ref[...]
              + br1_ref[...])
        h2 = jnp.maximum(_dot_t(h1, wl1_ref[...]) + bl1_ref[...], 0.0)
        y2_ref[...] = _dot_t(h2, wr2_ref[...])
        hr2_ref[...] = _dot_t(h2, wo2_ref[...]) + br2_ref[...]

    return pl.pallas_call(
        body,
        out_shape=[jax.ShapeDtypeStruct((N, W_rel2.shape[0]), jnp.float32),
                   jax.ShapeDtypeStruct((N, W_root2.shape[0]), jnp.float32)],
    )(part1, xr1, b_rel1.reshape(1, -1), W_l1, b_l1.reshape(1, -1),
      W_rel2, W_root2, b_rel2.reshape(1, -1))


def _tc_post(part2, hr2, W_l2, b_l2):
    H3 = W_l2.shape[1]

    def body(p_ref, hr_ref, wl2_ref, bl2_ref, o_ref):
        h3 = p_ref[0, :N, :H3] + p_ref[1, :N, :H3] + hr_ref[...]
        logits = _dot_t(h3, wl2_ref[...]) + bl2_ref[...]
        m = jnp.max(logits, axis=1, keepdims=True)
        shifted = logits - m
        lse = jnp.log(jnp.sum(jnp.exp(shifted), axis=1, keepdims=True))
        o_ref[...] = shifted - lse

    return pl.pallas_call(
        body,
        out_shape=jax.ShapeDtypeStruct((N, W_l2.shape[0]), jnp.float32),
    )(part2, hr2, W_l2, b_l2.reshape(1, -1))


def kernel(x, edge_index, W_rel1, b_rel1, W_root1, W_l1, b_l1,
           W_rel2, b_rel2, W_root2, W_l2, b_l2):
    y1, xr1 = _tc_pre(x, W_rel1, W_root1)
    part1 = _sc_segment_sum(y1, edge_index)
    y2, hr2 = _tc_mid(part1, xr1, b_rel1, W_l1, b_l1, W_rel2, W_root2, b_rel2)
    part2 = _sc_segment_sum(y2, edge_index)
    return _tc_post(part2, hr2, W_l2, b_l2)
